# baseline (speedup 1.0000x reference)
import math
import functools

import jax
import jax.numpy as jnp
from jax import lax
from jax.experimental import pallas as pl
from jax.experimental.pallas import tpu as pltpu
from jax.experimental.pallas import tpu_sc as plsc

D_MODEL = 1024
N_HEADS = 16
HEAD_DIM = 64
SPAN = 2048
N_EXP = 16
D_FF = 2048
MU = 0.9
GAMMA = 1.0
M = 2048
LTOT = SPAN + M
P = 2 * M
NB = P // 512
NU = NB + N_EXP - 1

BQ = 256
W = BQ + SPAN
BLK = 512
NEG = -1e30
SW = 128

NC = 2
NS = 16
NW = NC * NS
PW = P // NW
HALF = PW // 2
TW = M // NW


def _ln(x, w, b):
    mu = jnp.mean(x, axis=-1, keepdims=True)
    var = jnp.mean((x - mu) ** 2, axis=-1, keepdims=True)
    return (x - mu) / jnp.sqrt(var + 1e-5) * w + b


def _dot_t(x, w):
    return lax.dot_general(x, w, (((1,), (1,)), ((), ())),
                           preferred_element_type=jnp.float32)


def _proj_qkv_kernel(x_ref, wq_ref, wkv_ref, q_ref, k_ref, v_ref):
    rb = pl.program_id(0)
    x = x_ref[...]
    y = _dot_t(x, wkv_ref[...])
    for h in range(N_HEADS):
        k_ref[h] = y[:, h * HEAD_DIM:(h + 1) * HEAD_DIM]
        v_ref[h] = y[:, D_MODEL + h * HEAD_DIM:D_MODEL + (h + 1) * HEAD_DIM]

    @pl.when(rb >= (LTOT - M) // BLK)
    def _():
        yq = _dot_t(x, wq_ref[...])
        for h in range(N_HEADS):
            q_ref[h] = yq[:, h * HEAD_DIM:(h + 1) * HEAD_DIM]


def _proj_qkv(x, wq, wkv):
    qoff = (LTOT - M) // BLK
    return pl.pallas_call(
        _proj_qkv_kernel,
        grid=(LTOT // BLK,),
        in_specs=[pl.BlockSpec((BLK, D_MODEL), lambda i: (i, 0)),
                  pl.BlockSpec((D_MODEL, D_MODEL), lambda i: (0, 0)),
                  pl.BlockSpec((2 * D_MODEL, D_MODEL), lambda i: (0, 0))],
        out_specs=[
            pl.BlockSpec((N_HEADS, BLK, HEAD_DIM),
                         lambda i: (0, jnp.maximum(i - qoff, 0), 0)),
            pl.BlockSpec((N_HEADS, BLK, HEAD_DIM), lambda i: (0, i, 0)),
            pl.BlockSpec((N_HEADS, BLK, HEAD_DIM), lambda i: (0, i, 0))],
        out_shape=[jax.ShapeDtypeStruct((N_HEADS, M, HEAD_DIM), jnp.float32),
                   jax.ShapeDtypeStruct((N_HEADS, LTOT, HEAD_DIM), jnp.float32),
                   jax.ShapeDtypeStruct((N_HEADS, LTOT, HEAD_DIM), jnp.float32)],
    )(x, wq, wkv)


GW = SPAN + 8


def _attn_kernel(q_ref, k_ref, v_ref, pos_ref, mask_ref, o_ref):
    qb = pl.program_id(1)
    r0 = qb * BQ
    q = q_ref[0]
    ks = k_ref[0, pl.ds(r0, W), :]
    vs = v_ref[0, pl.ds(r0, W), :]
    s = _dot_t(q, ks)
    rp = jnp.dot(q, pos_ref[...], preferred_element_type=jnp.float32)
    sh = jnp.concatenate([rp, jnp.zeros((BQ, BQ), jnp.float32)], axis=1)
    sh = pltpu.roll(sh, 0, 1, stride=1, stride_axis=0)
    s = (s + sh) * (1.0 / math.sqrt(D_MODEL)) + mask_ref[...]
    m = jnp.max(s, axis=-1, keepdims=True)
    p = jnp.exp(s - m)
    p = p / jnp.sum(p, axis=-1, keepdims=True)
    o_ref[0] = jnp.dot(p, vs, preferred_element_type=jnp.float32)


def _attention(qh, kh, vh, pos):
    ri = jnp.arange(BQ, dtype=jnp.int32)[:, None]
    ci = jnp.arange(W, dtype=jnp.int32)[None, :]
    mask_add = jnp.where((ci >= ri) & (ci < ri + SPAN), 0.0, NEG)
    return pl.pallas_call(
        _attn_kernel,
        grid=(N_HEADS, M // BQ),
        in_specs=[
            pl.BlockSpec((1, BQ, HEAD_DIM), lambda h, qb: (h, qb, 0)),
            pl.BlockSpec((1, LTOT, HEAD_DIM), lambda h, qb: (h, 0, 0)),
            pl.BlockSpec((1, LTOT, HEAD_DIM), lambda h, qb: (h, 0, 0)),
            pl.BlockSpec((HEAD_DIM, SPAN), lambda h, qb: (0, 0)),
            pl.BlockSpec((BQ, W), lambda h, qb: (0, 0)),
        ],
        out_specs=pl.BlockSpec((1, BQ, HEAD_DIM), lambda h, qb: (h, qb, 0)),
        out_shape=jax.ShapeDtypeStruct((N_HEADS, M, HEAD_DIM), jnp.float32),
    )(qh, kh, vh, pos, mask_add)


def _outproj_ln_kernel(ctx_ref, wo_ref, h_ref, w_ref, b_ref, o_ref):
    x = jnp.concatenate([ctx_ref[h] for h in range(N_HEADS)], axis=1)
    y = _dot_t(x, wo_ref[...]) + h_ref[...]
    o_ref[...] = _ln(y, w_ref[...], b_ref[...])


def _outproj_ln(ctx, wo, h2d, lnw, lnb):
    return pl.pallas_call(
        _outproj_ln_kernel,
        grid=(M // BLK,),
        in_specs=[pl.BlockSpec((N_HEADS, BLK, HEAD_DIM), lambda i: (0, i, 0)),
                  pl.BlockSpec((D_MODEL, D_MODEL), lambda i: (0, 0)),
                  pl.BlockSpec((BLK, D_MODEL), lambda i: (i, 0)),
                  pl.BlockSpec((1, D_MODEL), lambda i: (0, 0)),
                  pl.BlockSpec((1, D_MODEL), lambda i: (0, 0))],
        out_specs=pl.BlockSpec((BLK, D_MODEL), lambda i: (i, 0)),
        out_shape=jax.ShapeDtypeStruct((M, D_MODEL), jnp.float32),
    )(ctx, wo, h2d, lnw, lnb)


def _route_kernel(x_ref, gw_ref, gb_ref, pos_ref, srep_ref, cnt_ref):
    logits = _dot_t(x_ref[...], gw_ref[...]) + gb_ref[...]
    e_iota = lax.broadcasted_iota(jnp.int32, (M, N_EXP), 1)
    m1 = jnp.max(logits, axis=-1, keepdims=True)
    i1 = jnp.min(jnp.where(logits == m1, e_iota, N_EXP), axis=-1, keepdims=True)
    masked = jnp.where(e_iota == i1, NEG, logits)
    m2 = jnp.max(masked, axis=-1, keepdims=True)
    i2 = jnp.min(jnp.where(masked == m2, e_iota, N_EXP), axis=-1, keepdims=True)
    s1 = 1.0 / (1.0 + jnp.exp(m2 - m1))
    s2 = 1.0 - s1
    srep_ref[...] = jnp.concatenate(
        [jnp.broadcast_to(s1, (M, SW)), jnp.broadcast_to(s2, (M, SW))],
        axis=1)
    oh1 = jnp.where(i1 == e_iota, 1.0, 0.0)
    oh2 = jnp.where(i2 == e_iota, 1.0, 0.0)
    r_i = lax.broadcasted_iota(jnp.int32, (M, M), 0)
    c_i = lax.broadcasted_iota(jnp.int32, (M, M), 1)
    tril = jnp.where(c_i < r_i, 1.0, 0.0)
    cum = jnp.dot(tril, oh1 + oh2, preferred_element_type=jnp.float32)
    cnt = jnp.sum(oh1 + oh2, axis=0, keepdims=True)
    cnt_ref[...] = cnt
    inc = cnt
    for sft in (1, 2, 4, 8):
        inc = inc + jnp.concatenate(
            [jnp.zeros((1, sft), jnp.float32), inc[:, :N_EXP - sft]], axis=1)
    offs = inc - cnt
    base1 = jnp.sum((cum + offs) * oh1, axis=1, keepdims=True)
    base2 = jnp.sum((cum + offs + oh1) * oh2, axis=1, keepdims=True)
    pos_ref[...] = jnp.concatenate([base1, base2], axis=1).astype(jnp.int32)


def _route(h1, gw, gb):
    return pl.pallas_call(
        _route_kernel,
        grid=(1,),
        in_specs=[pl.BlockSpec((M, D_MODEL), lambda i: (0, 0)),
                  pl.BlockSpec((N_EXP, D_MODEL), lambda i: (0, 0)),
                  pl.BlockSpec((1, N_EXP), lambda i: (0, 0))],
        out_specs=[pl.BlockSpec((M, 2), lambda i: (0, 0)),
                   pl.BlockSpec((M, 2 * SW), lambda i: (0, 0)),
                   pl.BlockSpec((1, N_EXP), lambda i: (0, 0))],
        out_shape=[jax.ShapeDtypeStruct((M, 2), jnp.int32),
                   jax.ShapeDtypeStruct((M, 2 * SW), jnp.float32),
                   jax.ShapeDtypeStruct((1, N_EXP), jnp.float32)],
    )(h1, gw, gb)


@functools.cache
def _sc_mesh():
    return plsc.VectorSubcoreMesh(core_axis_name="c", subcore_axis_name="s",
                                  num_cores=NC, num_subcores=NS)


def _sc_dispatch(h1, pos3d, srep):
    @functools.partial(
        pl.kernel,
        out_type=[jax.ShapeDtypeStruct((P, D_MODEL), jnp.float32),
                  jax.ShapeDtypeStruct((P, SW), jnp.float32)],
        mesh=_sc_mesh(),
        scratch_types=[pltpu.VMEM((2, HALF), jnp.int32),
                       pltpu.VMEM((HALF,), jnp.int32),
                       pltpu.VMEM((HALF, D_MODEL), jnp.float32),
                       pltpu.VMEM((HALF, SW), jnp.float32),
                       pltpu.SemaphoreType.DMA],
    )
    def body(h1_hbm, pos_hbm, srep_hbm, xs_hbm, ss_hbm,
             idx_v, tok_v, rows_v, s_v, sem):
        wid = lax.axis_index("s") * NC + lax.axis_index("c")
        base = wid * PW
        pltpu.sync_copy(pos_hbm.at[wid], idx_v)
        for half in range(2):
            hb = base + half * HALF
            for c in range(HALF // NS):
                v = hb + c * NS + lax.broadcasted_iota(jnp.int32, (NS,), 0)
                tok_v[pl.ds(c * NS, NS)] = lax.shift_right_logical(v, 1)
            pltpu.async_copy(h1_hbm.at[tok_v], rows_v, sem).wait()
            pltpu.async_copy(rows_v, xs_hbm.at[idx_v.at[half]], sem).wait()
            pltpu.sync_copy(srep_hbm.at[pl.ds(hb, HALF)], s_v)
            pltpu.async_copy(s_v, ss_hbm.at[idx_v.at[half]], sem).wait()

    return body(h1, pos3d, srep)


def _gmm_kernel(work_ref, offs_ref, x_ref, w1_ref, b1_ref, w2_ref, b2_ref,
                s_ref, o_ref):
    u = pl.program_id(0)
    e = work_ref[0, u]
    b = work_ref[1, u]
    ok = work_ref[2, u]
    lo = jnp.maximum(offs_ref[e], b * BLK)
    hi = jnp.minimum(offs_ref[e + 1], (b + 1) * BLK)
    x = x_ref[...].astype(jnp.bfloat16)
    t = jnp.maximum(_dot_t(x, w1_ref[0]) + b1_ref[0], 0.0)
    y = (_dot_t(t.astype(jnp.bfloat16), w2_ref[0]) + b2_ref[0]) * s_ref[...][:, :1]
    row = lax.broadcasted_iota(jnp.int32, (BLK, D_MODEL), 0) + b * BLK
    mask = (row >= lo) & (row < hi) & (ok > 0)
    y = jnp.where(mask, y, 0.0)
    prev_b = work_ref[1, jnp.maximum(u - 1, 0)]
    first = (u == 0) | (b != prev_b)

    @pl.when(first)
    def _():
        o_ref[...] = y

    @pl.when(jnp.logical_not(first))
    def _():
        o_ref[...] += y


def _gmm(work, offs, xs, ew1, eb1, ew2, eb2, ss):
    grid_spec = pltpu.PrefetchScalarGridSpec(
        num_scalar_prefetch=2,
        grid=(NU,),
        in_specs=[
            pl.BlockSpec((BLK, D_MODEL), lambda u, wk, of: (wk[1, u], 0)),
            pl.BlockSpec((1, D_FF, D_MODEL), lambda u, wk, of: (wk[0, u], 0, 0)),
            pl.BlockSpec((1, 1, D_FF), lambda u, wk, of: (wk[0, u], 0, 0)),
            pl.BlockSpec((1, D_MODEL, D_FF), lambda u, wk, of: (wk[0, u], 0, 0)),
            pl.BlockSpec((1, 1, D_MODEL), lambda u, wk, of: (wk[0, u], 0, 0)),
            pl.BlockSpec((BLK, SW), lambda u, wk, of: (wk[1, u], 0)),
        ],
        out_specs=pl.BlockSpec((BLK, D_MODEL), lambda u, wk, of: (wk[1, u], 0)),
    )
    return pl.pallas_call(
        _gmm_kernel,
        grid_spec=grid_spec,
        out_shape=jax.ShapeDtypeStruct((P, D_MODEL), jnp.float32),
    )(work, offs, xs, ew1, eb1.reshape(N_EXP, 1, D_FF), ew2,
      eb2.reshape(N_EXP, 1, D_MODEL), ss)


def _sc_combine(yw, pe, po):
    @functools.partial(
        pl.kernel,
        out_type=[jax.ShapeDtypeStruct((M, D_MODEL), jnp.float32),
                  jax.ShapeDtypeStruct((M, D_MODEL), jnp.float32)],
        mesh=_sc_mesh(),
        scratch_types=[pltpu.VMEM((TW,), jnp.int32),
                       pltpu.VMEM((TW, D_MODEL), jnp.float32),
                       pltpu.SemaphoreType.DMA],
    )
    def body(yw_hbm, pe_hbm, po_hbm, a_hbm, b_hbm, idx_v, rows_v, sem):
        wid = lax.axis_index("s") * NC + lax.axis_index("c")
        base = wid * TW
        pltpu.sync_copy(pe_hbm.at[pl.ds(base, TW)], idx_v)
        pltpu.async_copy(yw_hbm.at[idx_v], rows_v, sem).wait()
        pltpu.sync_copy(rows_v, a_hbm.at[pl.ds(base, TW)])
        pltpu.sync_copy(po_hbm.at[pl.ds(base, TW)], idx_v)
        pltpu.async_copy(yw_hbm.at[idx_v], rows_v, sem).wait()
        pltpu.sync_copy(rows_v, b_hbm.at[pl.ds(base, TW)])

    return body(yw, pe, po)


def _tail_kernel(mom_ref, a_ref, b_ref, h1_ref, l2w_ref, l2b_ref,
                 w1_ref, b1_ref, w2_ref, b2_ref, l3w_ref, l3b_ref,
                 nm_ref, o_ref):
    nm = MU * mom_ref[...] + GAMMA * (a_ref[...] + b_ref[...])
    nm_ref[...] = nm
    x = _ln(2.0 * h1_ref[...] - nm, l2w_ref[...], l2b_ref[...])
    t = jnp.maximum(_dot_t(x.astype(jnp.bfloat16), w1_ref[...]) + b1_ref[...],
                    0.0)
    y = _dot_t(t.astype(jnp.bfloat16), w2_ref[...]) + b2_ref[...]
    o_ref[...] = _ln(x + y, l3w_ref[...], l3b_ref[...])


def _tail(mom2d, moe_a, moe_b, h1, l2w, l2b, w1, b1, w2, b2, l3w, l3b):
    vec = pl.BlockSpec((1, D_MODEL), lambda i: (0, 0))
    big = pl.BlockSpec((BLK, D_MODEL), lambda i: (i, 0))
    return pl.pallas_call(
        _tail_kernel,
        grid=(M // BLK,),
        in_specs=[big, big, big, big, vec, vec,
                  pl.BlockSpec((D_FF, D_MODEL), lambda i: (0, 0)),
                  pl.BlockSpec((1, D_FF), lambda i: (0, 0)),
                  pl.BlockSpec((D_MODEL, D_FF), lambda i: (0, 0)),
                  vec, vec, vec],
        out_specs=[big, big],
        out_shape=[jax.ShapeDtypeStruct((M, D_MODEL), jnp.float32),
                   jax.ShapeDtypeStruct((M, D_MODEL), jnp.float32)],
    )(mom2d, moe_a, moe_b, h1, l2w, l2b, w1, b1, w2, b2, l3w, l3b)


def _worklist(cnt):
    counts = cnt.reshape(N_EXP).astype(jnp.int32)
    offs = jnp.concatenate(
        [jnp.zeros((1,), jnp.int32), jnp.cumsum(counts)])
    first_blk = offs[:N_EXP] // BLK
    last_blk = jnp.maximum((offs[1:] - 1) // BLK, first_blk)
    units_e = jnp.where(counts > 0, last_blk - first_blk + 1, 0)
    cum_inc = jnp.cumsum(units_e)
    cum_exc = cum_inc - units_e
    u = jnp.arange(NU)
    e_u = jnp.sum((u[:, None] >= cum_inc[None, :]).astype(jnp.int32), axis=1)
    valid = e_u < N_EXP
    e_c = jnp.minimum(e_u, N_EXP - 1)
    b_u = first_blk[e_c] + (u - cum_exc[e_c])
    e_last = jnp.max(jnp.where(counts > 0, jnp.arange(N_EXP), -1))
    e_c = jnp.where(valid, e_c, e_last)
    b_u = jnp.where(valid, b_u, NB - 1)
    work = jnp.stack([e_c, b_u, valid.astype(jnp.int32)]).astype(jnp.int32)
    return work, offs


def kernel(h, h_cache, pos_encoding, momentum, Wq, Wk, Wv, Wo,
           ln1_w, ln1_b, ln2_w, ln2_b, ln3_w, ln3_b,
           gate_w, gate_b, ew1, eb1, ew2, eb2,
           ff_w1, ff_b1, ff_w2, ff_b2):
    h2d = h.reshape(M, D_MODEL)
    h_all = jnp.concatenate([h_cache.reshape(SPAN, D_MODEL), h2d], axis=0)

    wkv = jnp.concatenate([Wk, Wv], axis=0)
    qh, kh, vh = _proj_qkv(h_all, Wq, wkv)

    ctx = _attention(qh, kh, vh, pos_encoding)

    h1 = _outproj_ln(ctx, Wo, h2d, ln1_w.reshape(1, -1), ln1_b.reshape(1, -1))

    posM2, srep, cnt = _route(h1, gate_w, gate_b.reshape(1, -1))

    xs, ss = _sc_dispatch(h1, posM2.reshape(NW, 2, HALF), srep.reshape(P, SW))

    work, offs = _worklist(cnt)
    yw = _gmm(work, offs, xs, ew1.astype(jnp.bfloat16), eb1,
              ew2.astype(jnp.bfloat16), eb2, ss)

    moe_a, moe_b = _sc_combine(yw, posM2[:, 0], posM2[:, 1])

    new_mom, h3 = _tail(momentum.reshape(M, D_MODEL), moe_a, moe_b, h1,
                        ln2_w.reshape(1, -1), ln2_b.reshape(1, -1),
                        ff_w1.astype(jnp.bfloat16), ff_b1.reshape(1, -1),
                        ff_w2.astype(jnp.bfloat16),
                        ff_b2.reshape(1, -1),
                        ln3_w.reshape(1, -1), ln3_b.reshape(1, -1))

    return (h3.reshape(1, M, D_MODEL), new_mom.reshape(1, M, D_MODEL))


# speedup vs baseline: 1.1765x; 1.1765x over previous
import math
import functools

import jax
import jax.numpy as jnp
from jax import lax
from jax.experimental import pallas as pl
from jax.experimental.pallas import tpu as pltpu
from jax.experimental.pallas import tpu_sc as plsc

D_MODEL = 1024
N_HEADS = 16
HEAD_DIM = 64
SPAN = 2048
N_EXP = 16
D_FF = 2048
MU = 0.9
GAMMA = 1.0
M = 2048
LTOT = SPAN + M
P = 2 * M
GBLK = 256
NB = P // GBLK
NU = NB + N_EXP - 1

BQ = 256
W = BQ + SPAN
BLK = 512
NEG = -1e30
SW = 128

NC = 2
NS = 16
NW = NC * NS
PW = P // NW
HALF = PW // 2
TW = M // NW


def _ln(x, w, b):
    mu = jnp.mean(x, axis=-1, keepdims=True)
    var = jnp.mean((x - mu) ** 2, axis=-1, keepdims=True)
    return (x - mu) / jnp.sqrt(var + 1e-5) * w + b


def _dot_t(x, w):
    return lax.dot_general(x, w, (((1,), (1,)), ((), ())),
                           preferred_element_type=jnp.float32)


def _proj_qkv_kernel(x_ref, wq_ref, wkv_ref, q_ref, k_ref, v_ref):
    rb = pl.program_id(0)
    x = x_ref[...]
    y = _dot_t(x, wkv_ref[...])
    for h in range(N_HEADS):
        k_ref[h] = y[:, h * HEAD_DIM:(h + 1) * HEAD_DIM]
        v_ref[h] = y[:, D_MODEL + h * HEAD_DIM:D_MODEL + (h + 1) * HEAD_DIM]

    @pl.when(rb >= (LTOT - M) // BLK)
    def _():
        yq = _dot_t(x, wq_ref[...])
        for h in range(N_HEADS):
            q_ref[h] = yq[:, h * HEAD_DIM:(h + 1) * HEAD_DIM]


def _proj_qkv(x, wq, wkv):
    qoff = (LTOT - M) // BLK
    return pl.pallas_call(
        _proj_qkv_kernel,
        grid=(LTOT // BLK,),
        in_specs=[pl.BlockSpec((BLK, D_MODEL), lambda i: (i, 0)),
                  pl.BlockSpec((D_MODEL, D_MODEL), lambda i: (0, 0)),
                  pl.BlockSpec((2 * D_MODEL, D_MODEL), lambda i: (0, 0))],
        out_specs=[
            pl.BlockSpec((N_HEADS, BLK, HEAD_DIM),
                         lambda i: (0, jnp.maximum(i - qoff, 0), 0)),
            pl.BlockSpec((N_HEADS, BLK, HEAD_DIM), lambda i: (0, i, 0)),
            pl.BlockSpec((N_HEADS, BLK, HEAD_DIM), lambda i: (0, i, 0))],
        out_shape=[jax.ShapeDtypeStruct((N_HEADS, M, HEAD_DIM), jnp.float32),
                   jax.ShapeDtypeStruct((N_HEADS, LTOT, HEAD_DIM), jnp.float32),
                   jax.ShapeDtypeStruct((N_HEADS, LTOT, HEAD_DIM), jnp.float32)],
    )(x, wq, wkv)


GW = SPAN + 8


def _attn_kernel(q_ref, k_ref, v_ref, pos_ref, mask_ref, o_ref):
    qb = pl.program_id(1)
    r0 = qb * BQ
    q = q_ref[0]
    ks = k_ref[0, pl.ds(r0, W), :]
    vs = v_ref[0, pl.ds(r0, W), :]
    s = _dot_t(q, ks)
    rp = jnp.dot(q, pos_ref[...], preferred_element_type=jnp.float32)
    sh = jnp.concatenate([rp, jnp.zeros((BQ, BQ), jnp.float32)], axis=1)
    sh = pltpu.roll(sh, 0, 1, stride=1, stride_axis=0)
    s = (s + sh) * (1.0 / math.sqrt(D_MODEL)) + mask_ref[...]
    m = jnp.max(s, axis=-1, keepdims=True)
    p = jnp.exp(s - m)
    p = p / jnp.sum(p, axis=-1, keepdims=True)
    o_ref[0] = jnp.dot(p, vs, preferred_element_type=jnp.float32)


def _attention(qh, kh, vh, pos):
    ri = jnp.arange(BQ, dtype=jnp.int32)[:, None]
    ci = jnp.arange(W, dtype=jnp.int32)[None, :]
    mask_add = jnp.where((ci >= ri) & (ci < ri + SPAN), 0.0, NEG)
    return pl.pallas_call(
        _attn_kernel,
        grid=(N_HEADS, M // BQ),
        in_specs=[
            pl.BlockSpec((1, BQ, HEAD_DIM), lambda h, qb: (h, qb, 0)),
            pl.BlockSpec((1, LTOT, HEAD_DIM), lambda h, qb: (h, 0, 0)),
            pl.BlockSpec((1, LTOT, HEAD_DIM), lambda h, qb: (h, 0, 0)),
            pl.BlockSpec((HEAD_DIM, SPAN), lambda h, qb: (0, 0)),
            pl.BlockSpec((BQ, W), lambda h, qb: (0, 0)),
        ],
        out_specs=pl.BlockSpec((1, BQ, HEAD_DIM), lambda h, qb: (h, qb, 0)),
        out_shape=jax.ShapeDtypeStruct((N_HEADS, M, HEAD_DIM), jnp.float32),
    )(qh, kh, vh, pos, mask_add)


def _outproj_ln_kernel(ctx_ref, wo_ref, h_ref, w_ref, b_ref, o_ref):
    x = jnp.concatenate([ctx_ref[h] for h in range(N_HEADS)], axis=1)
    y = _dot_t(x, wo_ref[...]) + h_ref[...]
    o_ref[...] = _ln(y, w_ref[...], b_ref[...])


def _outproj_ln(ctx, wo, h2d, lnw, lnb):
    return pl.pallas_call(
        _outproj_ln_kernel,
        grid=(M // BLK,),
        in_specs=[pl.BlockSpec((N_HEADS, BLK, HEAD_DIM), lambda i: (0, i, 0)),
                  pl.BlockSpec((D_MODEL, D_MODEL), lambda i: (0, 0)),
                  pl.BlockSpec((BLK, D_MODEL), lambda i: (i, 0)),
                  pl.BlockSpec((1, D_MODEL), lambda i: (0, 0)),
                  pl.BlockSpec((1, D_MODEL), lambda i: (0, 0))],
        out_specs=pl.BlockSpec((BLK, D_MODEL), lambda i: (i, 0)),
        out_shape=jax.ShapeDtypeStruct((M, D_MODEL), jnp.float32),
    )(ctx, wo, h2d, lnw, lnb)


def _route_kernel(x_ref, gw_ref, gb_ref, pos_ref, srep_ref, cnt_ref):
    logits = _dot_t(x_ref[...], gw_ref[...]) + gb_ref[...]
    e_iota = lax.broadcasted_iota(jnp.int32, (M, N_EXP), 1)
    m1 = jnp.max(logits, axis=-1, keepdims=True)
    i1 = jnp.min(jnp.where(logits == m1, e_iota, N_EXP), axis=-1, keepdims=True)
    masked = jnp.where(e_iota == i1, NEG, logits)
    m2 = jnp.max(masked, axis=-1, keepdims=True)
    i2 = jnp.min(jnp.where(masked == m2, e_iota, N_EXP), axis=-1, keepdims=True)
    s1 = 1.0 / (1.0 + jnp.exp(m2 - m1))
    s2 = 1.0 - s1
    srep_ref[...] = jnp.concatenate(
        [jnp.broadcast_to(s1, (M, SW)), jnp.broadcast_to(s2, (M, SW))],
        axis=1)
    oh1 = jnp.where(i1 == e_iota, 1.0, 0.0)
    oh2 = jnp.where(i2 == e_iota, 1.0, 0.0)
    r_i = lax.broadcasted_iota(jnp.int32, (M, M), 0)
    c_i = lax.broadcasted_iota(jnp.int32, (M, M), 1)
    tril = jnp.where(c_i < r_i, 1.0, 0.0)
    cum = jnp.dot(tril, oh1 + oh2, preferred_element_type=jnp.float32)
    cnt = jnp.sum(oh1 + oh2, axis=0, keepdims=True)
    cnt_ref[...] = cnt
    inc = cnt
    for sft in (1, 2, 4, 8):
        inc = inc + jnp.concatenate(
            [jnp.zeros((1, sft), jnp.float32), inc[:, :N_EXP - sft]], axis=1)
    offs = inc - cnt
    base1 = jnp.sum((cum + offs) * oh1, axis=1, keepdims=True)
    base2 = jnp.sum((cum + offs + oh1) * oh2, axis=1, keepdims=True)
    pos_ref[...] = jnp.concatenate([base1, base2], axis=1).astype(jnp.int32)


def _route(h1, gw, gb):
    return pl.pallas_call(
        _route_kernel,
        grid=(1,),
        in_specs=[pl.BlockSpec((M, D_MODEL), lambda i: (0, 0)),
                  pl.BlockSpec((N_EXP, D_MODEL), lambda i: (0, 0)),
                  pl.BlockSpec((1, N_EXP), lambda i: (0, 0))],
        out_specs=[pl.BlockSpec((M, 2), lambda i: (0, 0)),
                   pl.BlockSpec((M, 2 * SW), lambda i: (0, 0)),
                   pl.BlockSpec((1, N_EXP), lambda i: (0, 0))],
        out_shape=[jax.ShapeDtypeStruct((M, 2), jnp.int32),
                   jax.ShapeDtypeStruct((M, 2 * SW), jnp.float32),
                   jax.ShapeDtypeStruct((1, N_EXP), jnp.float32)],
    )(h1, gw, gb)


@functools.cache
def _sc_mesh():
    return plsc.VectorSubcoreMesh(core_axis_name="c", subcore_axis_name="s",
                                  num_cores=NC, num_subcores=NS)


def _sc_dispatch(h1, pos3d, srep):
    @functools.partial(
        pl.kernel,
        out_type=[jax.ShapeDtypeStruct((P, D_MODEL), jnp.float32),
                  jax.ShapeDtypeStruct((P, SW), jnp.float32)],
        mesh=_sc_mesh(),
        scratch_types=[pltpu.VMEM((2, HALF), jnp.int32),
                       pltpu.VMEM((HALF,), jnp.int32),
                       pltpu.VMEM((HALF, D_MODEL), jnp.float32),
                       pltpu.VMEM((HALF, SW), jnp.float32),
                       pltpu.SemaphoreType.DMA],
    )
    def body(h1_hbm, pos_hbm, srep_hbm, xs_hbm, ss_hbm,
             idx_v, tok_v, rows_v, s_v, sem):
        wid = lax.axis_index("s") * NC + lax.axis_index("c")
        base = wid * PW
        pltpu.sync_copy(pos_hbm.at[wid], idx_v)
        for half in range(2):
            hb = base + half * HALF
            for c in range(HALF // NS):
                v = hb + c * NS + lax.broadcasted_iota(jnp.int32, (NS,), 0)
                tok_v[pl.ds(c * NS, NS)] = lax.shift_right_logical(v, 1)
            pltpu.async_copy(h1_hbm.at[tok_v], rows_v, sem).wait()
            pltpu.async_copy(rows_v, xs_hbm.at[idx_v.at[half]], sem).wait()
            pltpu.sync_copy(srep_hbm.at[pl.ds(hb, HALF)], s_v)
            pltpu.async_copy(s_v, ss_hbm.at[idx_v.at[half]], sem).wait()

    return body(h1, pos3d, srep)


def _gmm_kernel(work_ref, offs_ref, x_ref, w1_ref, b1_ref, w2_ref, b2_ref,
                s_ref, o_ref):
    u = pl.program_id(0)
    e = work_ref[0, u]
    b = work_ref[1, u]
    ok = work_ref[2, u]
    lo = jnp.maximum(offs_ref[e], b * GBLK)
    hi = jnp.minimum(offs_ref[e + 1], (b + 1) * GBLK)
    x = x_ref[...]
    t = jnp.maximum(_dot_t(x, w1_ref[0]) + b1_ref[0], 0.0)
    y = (_dot_t(t, w2_ref[0]) + b2_ref[0]) * s_ref[...][:, :1]
    row = lax.broadcasted_iota(jnp.int32, (GBLK, D_MODEL), 0) + b * GBLK
    mask = (row >= lo) & (row < hi) & (ok > 0)
    y = jnp.where(mask, y, 0.0)
    prev_b = work_ref[1, jnp.maximum(u - 1, 0)]
    first = (u == 0) | (b != prev_b)

    @pl.when(first)
    def _():
        o_ref[...] = y

    @pl.when(jnp.logical_not(first))
    def _():
        o_ref[...] += y


def _gmm(work, offs, xs, ew1, eb1, ew2, eb2, ss):
    grid_spec = pltpu.PrefetchScalarGridSpec(
        num_scalar_prefetch=2,
        grid=(NU,),
        in_specs=[
            pl.BlockSpec((GBLK, D_MODEL), lambda u, wk, of: (wk[1, u], 0)),
            pl.BlockSpec((1, D_FF, D_MODEL), lambda u, wk, of: (wk[0, u], 0, 0)),
            pl.BlockSpec((1, 1, D_FF), lambda u, wk, of: (wk[0, u], 0, 0)),
            pl.BlockSpec((1, D_MODEL, D_FF), lambda u, wk, of: (wk[0, u], 0, 0)),
            pl.BlockSpec((1, 1, D_MODEL), lambda u, wk, of: (wk[0, u], 0, 0)),
            pl.BlockSpec((GBLK, SW), lambda u, wk, of: (wk[1, u], 0)),
        ],
        out_specs=pl.BlockSpec((GBLK, D_MODEL), lambda u, wk, of: (wk[1, u], 0)),
    )
    return pl.pallas_call(
        _gmm_kernel,
        grid_spec=grid_spec,
        out_shape=jax.ShapeDtypeStruct((P, D_MODEL), jnp.float32),
    )(work, offs, xs, ew1, eb1.reshape(N_EXP, 1, D_FF), ew2,
      eb2.reshape(N_EXP, 1, D_MODEL), ss)


def _sc_combine(yw, pe, po):
    @functools.partial(
        pl.kernel,
        out_type=[jax.ShapeDtypeStruct((M, D_MODEL), jnp.float32),
                  jax.ShapeDtypeStruct((M, D_MODEL), jnp.float32)],
        mesh=_sc_mesh(),
        scratch_types=[pltpu.VMEM((TW,), jnp.int32),
                       pltpu.VMEM((TW, D_MODEL), jnp.float32),
                       pltpu.SemaphoreType.DMA],
    )
    def body(yw_hbm, pe_hbm, po_hbm, a_hbm, b_hbm, idx_v, rows_v, sem):
        wid = lax.axis_index("s") * NC + lax.axis_index("c")
        base = wid * TW
        pltpu.sync_copy(pe_hbm.at[pl.ds(base, TW)], idx_v)
        pltpu.async_copy(yw_hbm.at[idx_v], rows_v, sem).wait()
        pltpu.sync_copy(rows_v, a_hbm.at[pl.ds(base, TW)])
        pltpu.sync_copy(po_hbm.at[pl.ds(base, TW)], idx_v)
        pltpu.async_copy(yw_hbm.at[idx_v], rows_v, sem).wait()
        pltpu.sync_copy(rows_v, b_hbm.at[pl.ds(base, TW)])

    return body(yw, pe, po)


def _tail_kernel(mom_ref, a_ref, b_ref, h1_ref, l2w_ref, l2b_ref,
                 w1_ref, b1_ref, w2_ref, b2_ref, l3w_ref, l3b_ref,
                 nm_ref, o_ref):
    nm = MU * mom_ref[...] + GAMMA * (a_ref[...] + b_ref[...])
    nm_ref[...] = nm
    x = _ln(2.0 * h1_ref[...] - nm, l2w_ref[...], l2b_ref[...])
    t = jnp.maximum(_dot_t(x, w1_ref[...]) + b1_ref[...], 0.0)
    y = _dot_t(t, w2_ref[...]) + b2_ref[...]
    o_ref[...] = _ln(x + y, l3w_ref[...], l3b_ref[...])


def _tail(mom2d, moe_a, moe_b, h1, l2w, l2b, w1, b1, w2, b2, l3w, l3b):
    vec = pl.BlockSpec((1, D_MODEL), lambda i: (0, 0))
    big = pl.BlockSpec((BLK, D_MODEL), lambda i: (i, 0))
    return pl.pallas_call(
        _tail_kernel,
        grid=(M // BLK,),
        in_specs=[big, big, big, big, vec, vec,
                  pl.BlockSpec((D_FF, D_MODEL), lambda i: (0, 0)),
                  pl.BlockSpec((1, D_FF), lambda i: (0, 0)),
                  pl.BlockSpec((D_MODEL, D_FF), lambda i: (0, 0)),
                  vec, vec, vec],
        out_specs=[big, big],
        out_shape=[jax.ShapeDtypeStruct((M, D_MODEL), jnp.float32),
                   jax.ShapeDtypeStruct((M, D_MODEL), jnp.float32)],
    )(mom2d, moe_a, moe_b, h1, l2w, l2b, w1, b1, w2, b2, l3w, l3b)


def _worklist(cnt):
    counts = cnt.reshape(N_EXP).astype(jnp.int32)
    offs = jnp.concatenate(
        [jnp.zeros((1,), jnp.int32), jnp.cumsum(counts)])
    first_blk = offs[:N_EXP] // GBLK
    last_blk = jnp.maximum((offs[1:] - 1) // GBLK, first_blk)
    units_e = jnp.where(counts > 0, last_blk - first_blk + 1, 0)
    cum_inc = jnp.cumsum(units_e)
    cum_exc = cum_inc - units_e
    u = jnp.arange(NU)
    e_u = jnp.sum((u[:, None] >= cum_inc[None, :]).astype(jnp.int32), axis=1)
    valid = e_u < N_EXP
    e_c = jnp.minimum(e_u, N_EXP - 1)
    b_u = first_blk[e_c] + (u - cum_exc[e_c])
    e_last = jnp.max(jnp.where(counts > 0, jnp.arange(N_EXP), -1))
    e_c = jnp.where(valid, e_c, e_last)
    b_u = jnp.where(valid, b_u, NB - 1)
    work = jnp.stack([e_c, b_u, valid.astype(jnp.int32)]).astype(jnp.int32)
    return work, offs


def kernel(h, h_cache, pos_encoding, momentum, Wq, Wk, Wv, Wo,
           ln1_w, ln1_b, ln2_w, ln2_b, ln3_w, ln3_b,
           gate_w, gate_b, ew1, eb1, ew2, eb2,
           ff_w1, ff_b1, ff_w2, ff_b2):
    h2d = h.reshape(M, D_MODEL)
    h_all = jnp.concatenate([h_cache.reshape(SPAN, D_MODEL), h2d], axis=0)

    wkv = jnp.concatenate([Wk, Wv], axis=0)
    qh, kh, vh = _proj_qkv(h_all, Wq, wkv)

    ctx = _attention(qh, kh, vh, pos_encoding)

    h1 = _outproj_ln(ctx, Wo, h2d, ln1_w.reshape(1, -1), ln1_b.reshape(1, -1))

    posM2, srep, cnt = _route(h1, gate_w, gate_b.reshape(1, -1))

    xs, ss = _sc_dispatch(h1, posM2.reshape(NW, 2, HALF), srep.reshape(P, SW))

    work, offs = _worklist(cnt)
    yw = _gmm(work, offs, xs, ew1, eb1, ew2, eb2, ss)

    moe_a, moe_b = _sc_combine(yw, posM2[:, 0], posM2[:, 1])

    new_mom, h3 = _tail(momentum.reshape(M, D_MODEL), moe_a, moe_b, h1,
                        ln2_w.reshape(1, -1), ln2_b.reshape(1, -1),
                        ff_w1, ff_b1.reshape(1, -1), ff_w2,
                        ff_b2.reshape(1, -1),
                        ln3_w.reshape(1, -1), ln3_b.reshape(1, -1))

    return (h3.reshape(1, M, D_MODEL), new_mom.reshape(1, M, D_MODEL))


# speedup vs baseline: 1.1987x; 1.0189x over previous
import math
import functools

import jax
import jax.numpy as jnp
from jax import lax
from jax.experimental import pallas as pl
from jax.experimental.pallas import tpu as pltpu
from jax.experimental.pallas import tpu_sc as plsc

D_MODEL = 1024
N_HEADS = 16
HEAD_DIM = 64
SPAN = 2048
N_EXP = 16
D_FF = 2048
MU = 0.9
GAMMA = 1.0
M = 2048
LTOT = SPAN + M
P = 2 * M
GBLK = 512
NB = P // GBLK
NU = NB + N_EXP - 1

BQ = 256
W = BQ + SPAN
BLK = 512
NEG = -1e30
SW = 128

NC = 2
NS = 16
NW = NC * NS
PW = P // NW
HALF = PW // 2
TW = M // NW


def _ln(x, w, b):
    mu = jnp.mean(x, axis=-1, keepdims=True)
    var = jnp.mean((x - mu) ** 2, axis=-1, keepdims=True)
    return (x - mu) / jnp.sqrt(var + 1e-5) * w + b


def _dot_t(x, w):
    return lax.dot_general(x, w, (((1,), (1,)), ((), ())),
                           preferred_element_type=jnp.float32)


def _proj_qkv_kernel(x_ref, wq_ref, wkv_ref, q_ref, k_ref, v_ref):
    rb = pl.program_id(0)
    x = x_ref[...]
    y = _dot_t(x, wkv_ref[...])
    for h in range(N_HEADS):
        k_ref[h] = y[:, h * HEAD_DIM:(h + 1) * HEAD_DIM]
        v_ref[h] = y[:, D_MODEL + h * HEAD_DIM:D_MODEL + (h + 1) * HEAD_DIM]

    @pl.when(rb >= (LTOT - M) // BLK)
    def _():
        yq = _dot_t(x, wq_ref[...])
        for h in range(N_HEADS):
            q_ref[h] = yq[:, h * HEAD_DIM:(h + 1) * HEAD_DIM]


def _proj_qkv(x, wq, wkv):
    qoff = (LTOT - M) // BLK
    return pl.pallas_call(
        _proj_qkv_kernel,
        grid=(LTOT // BLK,),
        in_specs=[pl.BlockSpec((BLK, D_MODEL), lambda i: (i, 0)),
                  pl.BlockSpec((D_MODEL, D_MODEL), lambda i: (0, 0)),
                  pl.BlockSpec((2 * D_MODEL, D_MODEL), lambda i: (0, 0))],
        out_specs=[
            pl.BlockSpec((N_HEADS, BLK, HEAD_DIM),
                         lambda i: (0, jnp.maximum(i - qoff, 0), 0)),
            pl.BlockSpec((N_HEADS, BLK, HEAD_DIM), lambda i: (0, i, 0)),
            pl.BlockSpec((N_HEADS, BLK, HEAD_DIM), lambda i: (0, i, 0))],
        out_shape=[jax.ShapeDtypeStruct((N_HEADS, M, HEAD_DIM), jnp.float32),
                   jax.ShapeDtypeStruct((N_HEADS, LTOT, HEAD_DIM), jnp.float32),
                   jax.ShapeDtypeStruct((N_HEADS, LTOT, HEAD_DIM), jnp.float32)],
    )(x, wq, wkv)


GW = SPAN + 8


def _attn_kernel(q_ref, k_ref, v_ref, pos_ref, mask_ref, o_ref):
    qb = pl.program_id(1)
    r0 = qb * BQ
    q = q_ref[0] * (1.0 / math.sqrt(D_MODEL))
    ks = k_ref[0, pl.ds(r0, W), :]
    vs = v_ref[0, pl.ds(r0, W), :]
    s = _dot_t(q, ks)
    rp = jnp.dot(q, pos_ref[...], preferred_element_type=jnp.float32)
    sh = jnp.concatenate([rp, jnp.zeros((BQ, BQ), jnp.float32)], axis=1)
    sh = pltpu.roll(sh, 0, 1, stride=1, stride_axis=0)
    s = s + sh + mask_ref[...]
    m = jnp.max(s, axis=-1, keepdims=True)
    p = jnp.exp(s - m)
    z = jnp.sum(p, axis=-1, keepdims=True)
    o_ref[0] = jnp.dot(p, vs, preferred_element_type=jnp.float32) / z


def _attention(qh, kh, vh, pos):
    ri = jnp.arange(BQ, dtype=jnp.int32)[:, None]
    ci = jnp.arange(W, dtype=jnp.int32)[None, :]
    mask_add = jnp.where((ci >= ri) & (ci < ri + SPAN), 0.0, NEG)
    return pl.pallas_call(
        _attn_kernel,
        grid=(N_HEADS, M // BQ),
        in_specs=[
            pl.BlockSpec((1, BQ, HEAD_DIM), lambda h, qb: (h, qb, 0)),
            pl.BlockSpec((1, LTOT, HEAD_DIM), lambda h, qb: (h, 0, 0)),
            pl.BlockSpec((1, LTOT, HEAD_DIM), lambda h, qb: (h, 0, 0)),
            pl.BlockSpec((HEAD_DIM, SPAN), lambda h, qb: (0, 0)),
            pl.BlockSpec((BQ, W), lambda h, qb: (0, 0)),
        ],
        out_specs=pl.BlockSpec((1, BQ, HEAD_DIM), lambda h, qb: (h, qb, 0)),
        out_shape=jax.ShapeDtypeStruct((N_HEADS, M, HEAD_DIM), jnp.float32),
    )(qh, kh, vh, pos, mask_add)


def _outproj_ln_kernel(ctx_ref, wo_ref, h_ref, w_ref, b_ref, o_ref):
    x = jnp.concatenate([ctx_ref[h] for h in range(N_HEADS)], axis=1)
    y = _dot_t(x, wo_ref[...]) + h_ref[...]
    o_ref[...] = _ln(y, w_ref[...], b_ref[...])


def _outproj_ln(ctx, wo, h2d, lnw, lnb):
    return pl.pallas_call(
        _outproj_ln_kernel,
        grid=(M // BLK,),
        in_specs=[pl.BlockSpec((N_HEADS, BLK, HEAD_DIM), lambda i: (0, i, 0)),
                  pl.BlockSpec((D_MODEL, D_MODEL), lambda i: (0, 0)),
                  pl.BlockSpec((BLK, D_MODEL), lambda i: (i, 0)),
                  pl.BlockSpec((1, D_MODEL), lambda i: (0, 0)),
                  pl.BlockSpec((1, D_MODEL), lambda i: (0, 0))],
        out_specs=pl.BlockSpec((BLK, D_MODEL), lambda i: (i, 0)),
        out_shape=jax.ShapeDtypeStruct((M, D_MODEL), jnp.float32),
    )(ctx, wo, h2d, lnw, lnb)


def _route_kernel(x_ref, gw_ref, gb_ref, pos_ref, srep_ref, cnt_ref):
    logits = _dot_t(x_ref[...], gw_ref[...]) + gb_ref[...]
    e_iota = lax.broadcasted_iota(jnp.int32, (M, N_EXP), 1)
    m1 = jnp.max(logits, axis=-1, keepdims=True)
    i1 = jnp.min(jnp.where(logits == m1, e_iota, N_EXP), axis=-1, keepdims=True)
    masked = jnp.where(e_iota == i1, NEG, logits)
    m2 = jnp.max(masked, axis=-1, keepdims=True)
    i2 = jnp.min(jnp.where(masked == m2, e_iota, N_EXP), axis=-1, keepdims=True)
    s1 = 1.0 / (1.0 + jnp.exp(m2 - m1))
    s2 = 1.0 - s1
    srep_ref[...] = jnp.concatenate(
        [jnp.broadcast_to(s1, (M, SW)), jnp.broadcast_to(s2, (M, SW))],
        axis=1)
    oh1 = jnp.where(i1 == e_iota, 1.0, 0.0)
    oh2 = jnp.where(i2 == e_iota, 1.0, 0.0)
    r_i = lax.broadcasted_iota(jnp.int32, (M, M), 0)
    c_i = lax.broadcasted_iota(jnp.int32, (M, M), 1)
    tril = jnp.where(c_i < r_i, 1.0, 0.0)
    cum = jnp.dot(tril, oh1 + oh2, preferred_element_type=jnp.float32)
    cnt = jnp.sum(oh1 + oh2, axis=0, keepdims=True)
    cnt_ref[...] = cnt
    inc = cnt
    for sft in (1, 2, 4, 8):
        inc = inc + jnp.concatenate(
            [jnp.zeros((1, sft), jnp.float32), inc[:, :N_EXP - sft]], axis=1)
    offs = inc - cnt
    base1 = jnp.sum((cum + offs) * oh1, axis=1, keepdims=True)
    base2 = jnp.sum((cum + offs + oh1) * oh2, axis=1, keepdims=True)
    pos_ref[...] = jnp.concatenate([base1, base2], axis=1).astype(jnp.int32)


def _route(h1, gw, gb):
    return pl.pallas_call(
        _route_kernel,
        grid=(1,),
        in_specs=[pl.BlockSpec((M, D_MODEL), lambda i: (0, 0)),
                  pl.BlockSpec((N_EXP, D_MODEL), lambda i: (0, 0)),
                  pl.BlockSpec((1, N_EXP), lambda i: (0, 0))],
        out_specs=[pl.BlockSpec((M, 2), lambda i: (0, 0)),
                   pl.BlockSpec((M, 2 * SW), lambda i: (0, 0)),
                   pl.BlockSpec((1, N_EXP), lambda i: (0, 0))],
        out_shape=[jax.ShapeDtypeStruct((M, 2), jnp.int32),
                   jax.ShapeDtypeStruct((M, 2 * SW), jnp.float32),
                   jax.ShapeDtypeStruct((1, N_EXP), jnp.float32)],
    )(h1, gw, gb)


@functools.cache
def _sc_mesh():
    return plsc.VectorSubcoreMesh(core_axis_name="c", subcore_axis_name="s",
                                  num_cores=NC, num_subcores=NS)


def _sc_dispatch(h1, pos3d, srep):
    @functools.partial(
        pl.kernel,
        out_type=[jax.ShapeDtypeStruct((P, D_MODEL), jnp.float32),
                  jax.ShapeDtypeStruct((P, SW), jnp.float32)],
        mesh=_sc_mesh(),
        scratch_types=[pltpu.VMEM((2, HALF), jnp.int32),
                       pltpu.VMEM((HALF,), jnp.int32),
                       pltpu.VMEM((HALF, D_MODEL), jnp.float32),
                       pltpu.VMEM((HALF, SW), jnp.float32),
                       pltpu.SemaphoreType.DMA],
    )
    def body(h1_hbm, pos_hbm, srep_hbm, xs_hbm, ss_hbm,
             idx_v, tok_v, rows_v, s_v, sem):
        wid = lax.axis_index("s") * NC + lax.axis_index("c")
        base = wid * PW
        pltpu.sync_copy(pos_hbm.at[wid], idx_v)
        for half in range(2):
            hb = base + half * HALF
            for c in range(HALF // NS):
                v = hb + c * NS + lax.broadcasted_iota(jnp.int32, (NS,), 0)
                tok_v[pl.ds(c * NS, NS)] = lax.shift_right_logical(v, 1)
            pltpu.async_copy(h1_hbm.at[tok_v], rows_v, sem).wait()
            pltpu.async_copy(rows_v, xs_hbm.at[idx_v.at[half]], sem).wait()
            pltpu.sync_copy(srep_hbm.at[pl.ds(hb, HALF)], s_v)
            pltpu.async_copy(s_v, ss_hbm.at[idx_v.at[half]], sem).wait()

    return body(h1, pos3d, srep)


def _gmm_kernel(work_ref, offs_ref, x_ref, w1_ref, b1_ref, w2_ref, b2_ref,
                s_ref, o_ref):
    u = pl.program_id(0)
    e = work_ref[0, u]
    b = work_ref[1, u]
    ok = work_ref[2, u]
    lo = jnp.maximum(offs_ref[e], b * GBLK)
    hi = jnp.minimum(offs_ref[e + 1], (b + 1) * GBLK)
    x = x_ref[...]
    t = jnp.maximum(_dot_t(x, w1_ref[0]) + b1_ref[0], 0.0)
    y = (_dot_t(t, w2_ref[0]) + b2_ref[0]) * s_ref[...][:, :1]
    row = lax.broadcasted_iota(jnp.int32, (GBLK, D_MODEL), 0) + b * GBLK
    mask = (row >= lo) & (row < hi) & (ok > 0)
    y = jnp.where(mask, y, 0.0)
    prev_b = work_ref[1, jnp.maximum(u - 1, 0)]
    first = (u == 0) | (b != prev_b)

    @pl.when(first)
    def _():
        o_ref[...] = y

    @pl.when(jnp.logical_not(first))
    def _():
        o_ref[...] += y


def _gmm(work, offs, xs, ew1, eb1, ew2, eb2, ss):
    grid_spec = pltpu.PrefetchScalarGridSpec(
        num_scalar_prefetch=2,
        grid=(NU,),
        in_specs=[
            pl.BlockSpec((GBLK, D_MODEL), lambda u, wk, of: (wk[1, u], 0)),
            pl.BlockSpec((1, D_FF, D_MODEL), lambda u, wk, of: (wk[0, u], 0, 0)),
            pl.BlockSpec((1, 1, D_FF), lambda u, wk, of: (wk[0, u], 0, 0)),
            pl.BlockSpec((1, D_MODEL, D_FF), lambda u, wk, of: (wk[0, u], 0, 0)),
            pl.BlockSpec((1, 1, D_MODEL), lambda u, wk, of: (wk[0, u], 0, 0)),
            pl.BlockSpec((GBLK, SW), lambda u, wk, of: (wk[1, u], 0)),
        ],
        out_specs=pl.BlockSpec((GBLK, D_MODEL), lambda u, wk, of: (wk[1, u], 0)),
    )
    return pl.pallas_call(
        _gmm_kernel,
        grid_spec=grid_spec,
        out_shape=jax.ShapeDtypeStruct((P, D_MODEL), jnp.float32),
    )(work, offs, xs, ew1, eb1.reshape(N_EXP, 1, D_FF), ew2,
      eb2.reshape(N_EXP, 1, D_MODEL), ss)


def _sc_combine(yw, pe, po):
    @functools.partial(
        pl.kernel,
        out_type=[jax.ShapeDtypeStruct((M, D_MODEL), jnp.float32),
                  jax.ShapeDtypeStruct((M, D_MODEL), jnp.float32)],
        mesh=_sc_mesh(),
        scratch_types=[pltpu.VMEM((TW,), jnp.int32),
                       pltpu.VMEM((TW, D_MODEL), jnp.float32),
                       pltpu.SemaphoreType.DMA],
    )
    def body(yw_hbm, pe_hbm, po_hbm, a_hbm, b_hbm, idx_v, rows_v, sem):
        wid = lax.axis_index("s") * NC + lax.axis_index("c")
        base = wid * TW
        pltpu.sync_copy(pe_hbm.at[pl.ds(base, TW)], idx_v)
        pltpu.async_copy(yw_hbm.at[idx_v], rows_v, sem).wait()
        pltpu.sync_copy(rows_v, a_hbm.at[pl.ds(base, TW)])
        pltpu.sync_copy(po_hbm.at[pl.ds(base, TW)], idx_v)
        pltpu.async_copy(yw_hbm.at[idx_v], rows_v, sem).wait()
        pltpu.sync_copy(rows_v, b_hbm.at[pl.ds(base, TW)])

    return body(yw, pe, po)


def _tail_kernel(mom_ref, a_ref, b_ref, h1_ref, l2w_ref, l2b_ref,
                 w1_ref, b1_ref, w2_ref, b2_ref, l3w_ref, l3b_ref,
                 nm_ref, o_ref):
    nm = MU * mom_ref[...] + GAMMA * (a_ref[...] + b_ref[...])
    nm_ref[...] = nm
    x = _ln(2.0 * h1_ref[...] - nm, l2w_ref[...], l2b_ref[...])
    t = jnp.maximum(_dot_t(x, w1_ref[...]) + b1_ref[...], 0.0)
    y = _dot_t(t, w2_ref[...]) + b2_ref[...]
    o_ref[...] = _ln(x + y, l3w_ref[...], l3b_ref[...])


def _tail(mom2d, moe_a, moe_b, h1, l2w, l2b, w1, b1, w2, b2, l3w, l3b):
    vec = pl.BlockSpec((1, D_MODEL), lambda i: (0, 0))
    big = pl.BlockSpec((BLK, D_MODEL), lambda i: (i, 0))
    return pl.pallas_call(
        _tail_kernel,
        grid=(M // BLK,),
        in_specs=[big, big, big, big, vec, vec,
                  pl.BlockSpec((D_FF, D_MODEL), lambda i: (0, 0)),
                  pl.BlockSpec((1, D_FF), lambda i: (0, 0)),
                  pl.BlockSpec((D_MODEL, D_FF), lambda i: (0, 0)),
                  vec, vec, vec],
        out_specs=[big, big],
        out_shape=[jax.ShapeDtypeStruct((M, D_MODEL), jnp.float32),
                   jax.ShapeDtypeStruct((M, D_MODEL), jnp.float32)],
    )(mom2d, moe_a, moe_b, h1, l2w, l2b, w1, b1, w2, b2, l3w, l3b)


def _worklist(cnt):
    counts = cnt.reshape(N_EXP).astype(jnp.int32)
    offs = jnp.concatenate(
        [jnp.zeros((1,), jnp.int32), jnp.cumsum(counts)])
    first_blk = offs[:N_EXP] // GBLK
    last_blk = jnp.maximum((offs[1:] - 1) // GBLK, first_blk)
    units_e = jnp.where(counts > 0, last_blk - first_blk + 1, 0)
    cum_inc = jnp.cumsum(units_e)
    cum_exc = cum_inc - units_e
    u = jnp.arange(NU)
    e_u = jnp.sum((u[:, None] >= cum_inc[None, :]).astype(jnp.int32), axis=1)
    valid = e_u < N_EXP
    e_c = jnp.minimum(e_u, N_EXP - 1)
    b_u = first_blk[e_c] + (u - cum_exc[e_c])
    e_last = jnp.max(jnp.where(counts > 0, jnp.arange(N_EXP), -1))
    e_c = jnp.where(valid, e_c, e_last)
    b_u = jnp.where(valid, b_u, NB - 1)
    work = jnp.stack([e_c, b_u, valid.astype(jnp.int32)]).astype(jnp.int32)
    return work, offs


def kernel(h, h_cache, pos_encoding, momentum, Wq, Wk, Wv, Wo,
           ln1_w, ln1_b, ln2_w, ln2_b, ln3_w, ln3_b,
           gate_w, gate_b, ew1, eb1, ew2, eb2,
           ff_w1, ff_b1, ff_w2, ff_b2):
    h2d = h.reshape(M, D_MODEL)
    h_all = jnp.concatenate([h_cache.reshape(SPAN, D_MODEL), h2d], axis=0)

    wkv = jnp.concatenate([Wk, Wv], axis=0)
    qh, kh, vh = _proj_qkv(h_all, Wq, wkv)

    ctx = _attention(qh, kh, vh, pos_encoding)

    h1 = _outproj_ln(ctx, Wo, h2d, ln1_w.reshape(1, -1), ln1_b.reshape(1, -1))

    posM2, srep, cnt = _route(h1, gate_w, gate_b.reshape(1, -1))

    xs, ss = _sc_dispatch(h1, posM2.reshape(NW, 2, HALF), srep.reshape(P, SW))

    work, offs = _worklist(cnt)
    yw = _gmm(work, offs, xs, ew1, eb1, ew2, eb2, ss)

    moe_a, moe_b = _sc_combine(yw, posM2[:, 0], posM2[:, 1])

    new_mom, h3 = _tail(momentum.reshape(M, D_MODEL), moe_a, moe_b, h1,
                        ln2_w.reshape(1, -1), ln2_b.reshape(1, -1),
                        ff_w1, ff_b1.reshape(1, -1), ff_w2,
                        ff_b2.reshape(1, -1),
                        ln3_w.reshape(1, -1), ln3_b.reshape(1, -1))

    return (h3.reshape(1, M, D_MODEL), new_mom.reshape(1, M, D_MODEL))
